# Initial kernel scaffold; baseline (speedup 1.0000x reference)
import jax
import jax.numpy as jnp
from jax.experimental import pallas as pl


def kernel(hist_indices, hist_values, target_items, pi, beta_q, beta_k, delta_response, delta_plus_k, delta_minus_k):
    raise NotImplementedError("write your pallas kernel here")


# speedup vs baseline: 1.2103x; 1.2103x over previous
import functools
import math

import jax
import jax.numpy as jnp
from jax import lax
from jax.experimental import pallas as pl
from jax.experimental.pallas import tpu as pltpu
from jax.experimental.pallas import tpu_sc as plsc

NUM_ITEMS = 100000
E = NUM_ITEMS + 1
R = 64
B = 4096
H = 200

NC = 2
NS = 16
L = 16
NW = NC * NS
BPW = B // NW

HA = 128
HBC = 80
HP = HA + HBC
NBLK = HP // L
HPAD = 224

_NEG = -10000.0
_ISQ = 1.0 / math.sqrt(R)


def _sc_body(hidx_hbm, hval_hbm, tgt_hbm, bq_hbm, dresp_hbm, bk_hbm, dcat_hbm,
             out_hbm,
             tidx, qb, qd, hi2, hv2, ci2, bkrows, drrows, outbuf, sem0, sem1):
    cid = lax.axis_index("c")
    sid = lax.axis_index("s")
    wid = sid * NC + cid
    base = wid * BPW

    sems = (sem0, sem1)

    pltpu.sync_copy(tgt_hbm.at[pl.ds(base, BPW)], tidx)
    pltpu.async_copy(bq_hbm.at[tidx], qb, sem0).wait()
    pltpu.async_copy(dresp_hbm.at[tidx], qd, sem0).wait()

    def prep(r, buf):
        gb = base + r
        sem = sems[buf]
        pltpu.sync_copy(hidx_hbm.at[gb, pl.ds(0, HA)], hi2.at[buf, pl.ds(0, HA)])
        pltpu.sync_copy(hidx_hbm.at[gb, pl.ds(HA, HBC)], hi2.at[buf, pl.ds(HA, HBC)])
        pltpu.sync_copy(hval_hbm.at[gb, pl.ds(0, HA)], hv2.at[buf, pl.ds(0, HA)])
        pltpu.sync_copy(hval_hbm.at[gb, pl.ds(HA, HBC)], hv2.at[buf, pl.ds(HA, HBC)])
        for j in range(NBLK):
            hi = hi2[buf, pl.ds(j * L, L)]
            hv = hv2[buf, pl.ds(j * L, L)]
            ci = jnp.where(hv > 0.5, hi, jnp.where(hv < -0.5, hi + E, 0))
            ci2[buf, pl.ds(j * L, L)] = ci
        pltpu.async_copy(bk_hbm.at[hi2.at[buf, pl.ds(0, HA)]],
                         bkrows.at[buf, pl.ds(0, HA)], sem)
        pltpu.async_copy(bk_hbm.at[hi2.at[buf, pl.ds(HA, HBC)]],
                         bkrows.at[buf, pl.ds(HA, HBC)], sem)
        pltpu.async_copy(dcat_hbm.at[ci2.at[buf, pl.ds(0, HA)]],
                         drrows.at[buf, pl.ds(0, HA)], sem)
        pltpu.async_copy(dcat_hbm.at[ci2.at[buf, pl.ds(HA, HBC)]],
                         drrows.at[buf, pl.ds(HA, HBC)], sem)

    def wait(buf):
        sem = sems[buf]
        pltpu.make_async_copy(bk_hbm.at[hi2.at[buf, pl.ds(0, HA)]],
                              bkrows.at[buf, pl.ds(0, HA)], sem).wait()
        pltpu.make_async_copy(bk_hbm.at[hi2.at[buf, pl.ds(HA, HBC)]],
                              bkrows.at[buf, pl.ds(HA, HBC)], sem).wait()
        pltpu.make_async_copy(dcat_hbm.at[ci2.at[buf, pl.ds(0, HA)]],
                              drrows.at[buf, pl.ds(0, HA)], sem).wait()
        pltpu.make_async_copy(dcat_hbm.at[ci2.at[buf, pl.ds(HA, HBC)]],
                              drrows.at[buf, pl.ds(HA, HBC)], sem).wait()

    def dot_accumulate(rows, qref, r):
        lane = lax.iota(jnp.int32, L)
        rv = jnp.full((L,), r, jnp.int32)

        def body(rr, accs):
            rrv = jnp.full((L,), rr, jnp.int32)
            qsplat = plsc.load_gather(qref, [rv, rrv])
            out = []
            for j in range(NBLK):
                hvec = lane + (j * L)
                col = plsc.load_gather(rows, [hvec, rrv])
                out.append(accs[j] + qsplat * col)
            return tuple(out)

        zero = jnp.zeros((L,), jnp.float32)
        return lax.fori_loop(0, R, body, (zero,) * NBLK)

    def compute(r, buf):
        scores = dot_accumulate(bkrows.at[buf], qb, r)
        evs = dot_accumulate(drrows.at[buf], qd, r)
        s = []
        for j in range(NBLK):
            hi = hi2[buf, pl.ds(j * L, L)]
            s.append(jnp.where(hi != 0, scores[j] * _ISQ, _NEG))
        mx = s[0]
        for j in range(1, NBLK):
            mx = jnp.maximum(mx, s[j])
        mxs = jnp.max(mx)
        den = jnp.zeros((L,), jnp.float32)
        num = jnp.zeros((L,), jnp.float32)
        for j in range(NBLK):
            e = jnp.exp(s[j] - mxs)
            den = den + e
            num = num + e * evs[j]
        updv = jnp.full((L,), jnp.sum(num)) / jnp.full((L,), jnp.sum(den))
        lane = lax.iota(jnp.int32, L)
        plsc.store_scatter(outbuf, [jnp.full((L,), r, jnp.int32)],
                           updv, mask=lane == 0)

    prep(0, 0)

    def row_iter(it, carry):
        r0 = 2 * it
        prep(r0 + 1, 1)
        wait(0)
        compute(r0, 0)
        prep(jnp.minimum(r0 + 2, BPW - 1), 0)
        wait(1)
        compute(r0 + 1, 1)
        return carry

    lax.fori_loop(0, BPW // 2, row_iter, 0)
    wait(0)

    pltpu.sync_copy(outbuf, out_hbm.at[pl.ds(base, BPW)])


@functools.partial(jax.jit, static_argnames=())
def _prior(pi, target_items):
    p = pi[target_items - 1]
    p = jnp.clip(p, 1e-6, 1.0 - 1e-6)
    return jnp.log(p) - jnp.log1p(-p)


def kernel(hist_indices, hist_values, target_items, pi, beta_q, beta_k,
           delta_response, delta_plus_k, delta_minus_k):
    hidx = jnp.pad(hist_indices.astype(jnp.int32), ((0, 0), (0, HPAD - H)))
    hval = jnp.pad(hist_values, ((0, 0), (0, HPAD - H)))
    dcat = jnp.concatenate([delta_plus_k, delta_minus_k], axis=0)

    mesh = plsc.VectorSubcoreMesh(core_axis_name="c", subcore_axis_name="s")
    grid_kernel = pl.kernel(
        _sc_body,
        out_type=jax.ShapeDtypeStruct((B,), jnp.float32),
        mesh=mesh,
        compiler_params=pltpu.CompilerParams(needs_layout_passes=False,
                                             use_tc_tiling_on_sc=False),
        scratch_types=[
            pltpu.VMEM((BPW,), jnp.int32),
            pltpu.VMEM((BPW, R), jnp.float32),
            pltpu.VMEM((BPW, R), jnp.float32),
            pltpu.VMEM((2, HP), jnp.int32),
            pltpu.VMEM((2, HP), jnp.float32),
            pltpu.VMEM((2, HP), jnp.int32),
            pltpu.VMEM((2, HP, R), jnp.float32),
            pltpu.VMEM((2, HP, R), jnp.float32),
            pltpu.VMEM((BPW,), jnp.float32),
            pltpu.SemaphoreType.DMA,
            pltpu.SemaphoreType.DMA,
        ],
    )
    update = grid_kernel(hidx, hval, target_items.astype(jnp.int32),
                         beta_q, delta_response, beta_k, dcat)
    return _prior(pi, target_items) + update


# speedup vs baseline: 1.2109x; 1.0005x over previous
import functools
import math

import jax
import jax.numpy as jnp
from jax import lax
from jax.experimental import pallas as pl
from jax.experimental.pallas import tpu as pltpu
from jax.experimental.pallas import tpu_sc as plsc

NUM_ITEMS = 100000
E = NUM_ITEMS + 1
R = 64
B = 4096
H = 200

NC = 2
NS = 16
L = 16
NW = NC * NS
BPW = B // NW

HA = 128
HBC = 80
HP = HA + HBC
NBLK = HP // L

_NEG = -10000.0
_ISQ = 1.0 / math.sqrt(R)


def _sc_body(hidx_hbm, cidx_hbm, tgt_hbm, bq_hbm, dresp_hbm, bk_hbm, dcat_hbm,
             out_hbm,
             tidx, qb, qd, hi, ci, bkrows, drrows, outbuf, sem0, sem1):
    cid = lax.axis_index("c")
    sid = lax.axis_index("s")
    wid = sid * NC + cid
    base = wid * BPW

    sems = (sem0, sem1)

    pltpu.sync_copy(tgt_hbm.at[pl.ds(base, BPW)], tidx)
    pltpu.async_copy(bq_hbm.at[tidx], qb, sem0).wait()
    pltpu.async_copy(dresp_hbm.at[tidx], qd, sem0).wait()
    pltpu.sync_copy(hidx_hbm.at[pl.ds(base, BPW)], hi)
    pltpu.sync_copy(cidx_hbm.at[pl.ds(base, BPW)], ci)

    def prep(r, buf):
        sem = sems[buf]
        pltpu.async_copy(bk_hbm.at[hi.at[r, pl.ds(0, HA)]],
                         bkrows.at[buf, pl.ds(0, HA)], sem)
        pltpu.async_copy(bk_hbm.at[hi.at[r, pl.ds(HA, HBC)]],
                         bkrows.at[buf, pl.ds(HA, HBC)], sem)
        pltpu.async_copy(dcat_hbm.at[ci.at[r, pl.ds(0, HA)]],
                         drrows.at[buf, pl.ds(0, HA)], sem)
        pltpu.async_copy(dcat_hbm.at[ci.at[r, pl.ds(HA, HBC)]],
                         drrows.at[buf, pl.ds(HA, HBC)], sem)

    def wait(r, buf):
        sem = sems[buf]
        pltpu.make_async_copy(bk_hbm.at[hi.at[r, pl.ds(0, HA)]],
                              bkrows.at[buf, pl.ds(0, HA)], sem).wait()
        pltpu.make_async_copy(bk_hbm.at[hi.at[r, pl.ds(HA, HBC)]],
                              bkrows.at[buf, pl.ds(HA, HBC)], sem).wait()
        pltpu.make_async_copy(dcat_hbm.at[ci.at[r, pl.ds(0, HA)]],
                              drrows.at[buf, pl.ds(0, HA)], sem).wait()
        pltpu.make_async_copy(dcat_hbm.at[ci.at[r, pl.ds(HA, HBC)]],
                              drrows.at[buf, pl.ds(HA, HBC)], sem).wait()

    def dot_accumulate(rows, qref, r):
        lane = lax.iota(jnp.int32, L)
        rv = jnp.full((L,), r, jnp.int32)

        def body(rr, accs):
            rrv = jnp.full((L,), rr, jnp.int32)
            qsplat = plsc.load_gather(qref, [rv, rrv])
            out = []
            for j in range(NBLK):
                hvec = lane + (j * L)
                col = plsc.load_gather(rows, [hvec, rrv])
                out.append(accs[j] + qsplat * col)
            return tuple(out)

        zero = jnp.zeros((L,), jnp.float32)
        return lax.fori_loop(0, R, body, (zero,) * NBLK)

    def compute(r, buf):
        scores = dot_accumulate(bkrows.at[buf], qb, r)
        evs = dot_accumulate(drrows.at[buf], qd, r)
        s = []
        for j in range(NBLK):
            hij = hi[r, pl.ds(j * L, L)]
            s.append(jnp.where(hij != 0, scores[j] * _ISQ, _NEG))
        mx = s[0]
        for j in range(1, NBLK):
            mx = jnp.maximum(mx, s[j])
        mxs = jnp.max(mx)
        den = jnp.zeros((L,), jnp.float32)
        num = jnp.zeros((L,), jnp.float32)
        for j in range(NBLK):
            e = jnp.exp(s[j] - mxs)
            den = den + e
            num = num + e * evs[j]
        updv = jnp.full((L,), jnp.sum(num)) / jnp.full((L,), jnp.sum(den))
        lane = lax.iota(jnp.int32, L)
        plsc.store_scatter(outbuf, [jnp.full((L,), r, jnp.int32)],
                           updv, mask=lane == 0)

    prep(0, 0)

    def row_iter(it, carry):
        r0 = 2 * it
        prep(r0 + 1, 1)
        wait(r0, 0)
        compute(r0, 0)
        prep(jnp.minimum(r0 + 2, BPW - 1), 0)
        wait(r0 + 1, 1)
        compute(r0 + 1, 1)
        return carry

    lax.fori_loop(0, BPW // 2, row_iter, 0)
    wait(BPW - 1, 0)

    pltpu.sync_copy(outbuf, out_hbm.at[pl.ds(base, BPW)])


def kernel(hist_indices, hist_values, target_items, pi, beta_q, beta_k,
           delta_response, delta_plus_k, delta_minus_k):
    hidx = jnp.pad(hist_indices.astype(jnp.int32), ((0, 0), (0, HP - H)))
    cidx = jnp.where(hist_values > 0.5, hist_indices,
                     jnp.where(hist_values < -0.5, hist_indices + E, 0))
    cidx = jnp.pad(cidx.astype(jnp.int32), ((0, 0), (0, HP - H)))
    dcat = jnp.concatenate([delta_plus_k, delta_minus_k], axis=0)

    mesh = plsc.VectorSubcoreMesh(core_axis_name="c", subcore_axis_name="s")
    grid_kernel = pl.kernel(
        _sc_body,
        out_type=jax.ShapeDtypeStruct((B,), jnp.float32),
        mesh=mesh,
        compiler_params=pltpu.CompilerParams(needs_layout_passes=False,
                                             use_tc_tiling_on_sc=False),
        scratch_types=[
            pltpu.VMEM((BPW,), jnp.int32),
            pltpu.VMEM((BPW, R), jnp.float32),
            pltpu.VMEM((BPW, R), jnp.float32),
            pltpu.VMEM((BPW, HP), jnp.int32),
            pltpu.VMEM((BPW, HP), jnp.int32),
            pltpu.VMEM((2, HP, R), jnp.float32),
            pltpu.VMEM((2, HP, R), jnp.float32),
            pltpu.VMEM((BPW,), jnp.float32),
            pltpu.SemaphoreType.DMA,
            pltpu.SemaphoreType.DMA,
        ],
    )
    update = grid_kernel(hidx, cidx, target_items.astype(jnp.int32),
                         beta_q, delta_response, beta_k, dcat)
    p = pi[target_items - 1]
    p = jnp.clip(p, 1e-6, 1.0 - 1e-6)
    prior = jnp.log(p) - jnp.log1p(-p)
    return prior + update
